# Initial kernel scaffold; baseline (speedup 1.0000x reference)
import jax
import jax.numpy as jnp
from jax.experimental import pallas as pl


def kernel(x, scale, ln_bias, kernel, lora_a, lora_b, bias):
    raise NotImplementedError("write your pallas kernel here")


# speedup vs baseline: 1.2462x; 1.2462x over previous
import jax
import jax.numpy as jnp
from jax.experimental import pallas as pl
from jax.experimental.pallas import tpu as pltpu

_EPS = 1e-6
_BM = 256


def _fused_body(x_ref, scale_ref, lnb_ref, w_ref, a_ref, b_ref, bias_ref, o_ref):
    xv = x_ref[...]
    mean = jnp.mean(xv, axis=1, keepdims=True)
    xc = xv - mean
    var = jnp.mean(xc * xc, axis=1, keepdims=True)
    y = xc * jax.lax.rsqrt(var + _EPS) * scale_ref[...] + lnb_ref[...]
    yb = y.astype(jnp.bfloat16)
    z = jnp.dot(yb, a_ref[...], preferred_element_type=jnp.float32)
    acc = jnp.dot(yb, w_ref[...], preferred_element_type=jnp.float32)
    acc = acc + jnp.dot(z.astype(jnp.bfloat16), b_ref[...],
                        preferred_element_type=jnp.float32)
    o_ref[...] = acc + bias_ref[...]


def kernel(x, scale, ln_bias, kernel, lora_a, lora_b, bias):
    B, S, H = x.shape
    F = kernel.shape[1]
    M = B * S
    x2 = x.reshape(M, H)
    grid = (M // _BM,)

    out = pl.pallas_call(
        _fused_body,
        out_shape=jax.ShapeDtypeStruct((M, F), jnp.float32),
        grid=grid,
        in_specs=[
            pl.BlockSpec((_BM, H), lambda i: (i, 0)),
            pl.BlockSpec(memory_space=pltpu.VMEM),
            pl.BlockSpec(memory_space=pltpu.VMEM),
            pl.BlockSpec(memory_space=pltpu.VMEM),
            pl.BlockSpec(memory_space=pltpu.VMEM),
            pl.BlockSpec(memory_space=pltpu.VMEM),
            pl.BlockSpec(memory_space=pltpu.VMEM),
        ],
        out_specs=pl.BlockSpec((_BM, F), lambda i: (i, 0)),
        compiler_params=pltpu.CompilerParams(
            dimension_semantics=("parallel",),
            vmem_limit_bytes=56 * 1024 * 1024,
        ),
        name="ln_dense_lora",
    )(
        x2,
        scale.reshape(1, H),
        ln_bias.reshape(1, H),
        kernel.astype(jnp.bfloat16),
        lora_a.astype(jnp.bfloat16),
        lora_b.astype(jnp.bfloat16),
        bias.reshape(1, F),
    )
    return out.reshape(B, S, F)


# speedup vs baseline: 1.3743x; 1.1028x over previous
import jax
import jax.numpy as jnp
from jax.experimental import pallas as pl
from jax.experimental.pallas import tpu as pltpu

_EPS = 1e-6
_BM = 256
_BF = 512


def _fold_body(w_ref, a_ref, b_ref, o_ref):
    ab = jnp.dot(a_ref[...], b_ref[...], preferred_element_type=jnp.float32)
    o_ref[...] = (w_ref[...] + ab).astype(jnp.bfloat16)


def _main_body(x_ref, scale_ref, lnb_ref, w_ref, bias_ref, o_ref, y_scratch):
    i = pl.program_id(0)
    cur = jax.lax.rem(i, 2)
    prev = jax.lax.rem(i + 1, 2)

    acc = jnp.dot(y_scratch[prev], w_ref[...],
                  preferred_element_type=jnp.float32)
    o_ref[...] = acc + bias_ref[...]

    xv = x_ref[...]
    mean = jnp.mean(xv, axis=1, keepdims=True)
    xc = xv - mean
    var = jnp.mean(xc * xc, axis=1, keepdims=True)
    y = xc * jax.lax.rsqrt(var + _EPS) * scale_ref[...] + lnb_ref[...]
    y_scratch[cur] = y.astype(jnp.bfloat16)


def kernel(x, scale, ln_bias, kernel, lora_a, lora_b, bias):
    B, S, H = x.shape
    F = kernel.shape[1]
    R = lora_a.shape[1]
    M = B * S

    w_folded = pl.pallas_call(
        _fold_body,
        out_shape=jax.ShapeDtypeStruct((H, F), jnp.bfloat16),
        grid=(H // _BF,),
        in_specs=[
            pl.BlockSpec((_BF, F), lambda i: (i, 0)),
            pl.BlockSpec((_BF, R), lambda i: (i, 0)),
            pl.BlockSpec(memory_space=pltpu.VMEM),
        ],
        out_specs=pl.BlockSpec((_BF, F), lambda i: (i, 0)),
        compiler_params=pltpu.CompilerParams(
            dimension_semantics=("arbitrary",),
            vmem_limit_bytes=60000 * 1024,
        ),
        name="lora_fold",
    )(kernel, lora_a, lora_b)

    n_tiles = M // _BM
    x2 = x.reshape(M, H)

    out = pl.pallas_call(
        _main_body,
        out_shape=jax.ShapeDtypeStruct((M, F), jnp.float32),
        grid=(n_tiles + 1,),
        in_specs=[
            pl.BlockSpec((_BM, H), lambda i: (jnp.minimum(i, n_tiles - 1), 0)),
            pl.BlockSpec(memory_space=pltpu.VMEM),
            pl.BlockSpec(memory_space=pltpu.VMEM),
            pl.BlockSpec(memory_space=pltpu.VMEM),
            pl.BlockSpec(memory_space=pltpu.VMEM),
        ],
        out_specs=pl.BlockSpec((_BM, F),
                               lambda i: (jnp.maximum(i - 1, 0), 0)),
        scratch_shapes=[pltpu.VMEM((2, _BM, H), jnp.bfloat16)],
        compiler_params=pltpu.CompilerParams(
            dimension_semantics=("arbitrary",),
            vmem_limit_bytes=60000 * 1024,
        ),
        name="ln_dense",
    )(
        x2,
        scale.reshape(1, H),
        ln_bias.reshape(1, H),
        w_folded,
        bias.reshape(1, F),
    )
    return out.reshape(B, S, F)


# speedup vs baseline: 1.3919x; 1.0128x over previous
import jax
import jax.numpy as jnp
from jax.experimental import pallas as pl
from jax.experimental.pallas import tpu as pltpu

_EPS = 1e-6
_BM = 256
_BF = 512


def _fold_body(w_ref, a_ref, b_ref, scale_ref, lnb_ref, bias_ref,
               wout_ref, rb_ref):
    i = pl.program_id(0)
    wp = w_ref[...] + jnp.dot(a_ref[...], b_ref[...],
                              preferred_element_type=jnp.float32)
    wout_ref[...] = (wp * scale_ref[...]).astype(jnp.bfloat16)
    part = jnp.dot(lnb_ref[...], wp, preferred_element_type=jnp.float32)

    @pl.when(i == 0)
    def _():
        rb_ref[...] = bias_ref[...] + part

    @pl.when(i > 0)
    def _():
        rb_ref[...] = rb_ref[...] + part


def _main_body(x_ref, w_ref, rb_ref, o_ref, y_scratch):
    i = pl.program_id(0)
    cur = jax.lax.rem(i, 2)
    prev = jax.lax.rem(i + 1, 2)

    acc = jnp.dot(y_scratch[prev], w_ref[...],
                  preferred_element_type=jnp.float32)
    o_ref[...] = acc + rb_ref[...]

    xv = x_ref[...]
    inv_h = 1.0 / xv.shape[1]
    s1 = jnp.sum(xv, axis=1, keepdims=True)
    s2 = jnp.sum(xv * xv, axis=1, keepdims=True)
    mean = s1 * inv_h
    var = s2 * inv_h - mean * mean
    rstd = jax.lax.rsqrt(var + _EPS)
    y_scratch[cur] = ((xv - mean) * rstd).astype(jnp.bfloat16)


def kernel(x, scale, ln_bias, kernel, lora_a, lora_b, bias):
    B, S, H = x.shape
    F = kernel.shape[1]
    R = lora_a.shape[1]
    M = B * S

    w_folded, row_bias = pl.pallas_call(
        _fold_body,
        out_shape=(
            jax.ShapeDtypeStruct((H, F), jnp.bfloat16),
            jax.ShapeDtypeStruct((1, F), jnp.float32),
        ),
        grid=(H // _BF,),
        in_specs=[
            pl.BlockSpec((_BF, F), lambda i: (i, 0)),
            pl.BlockSpec((_BF, R), lambda i: (i, 0)),
            pl.BlockSpec(memory_space=pltpu.VMEM),
            pl.BlockSpec((_BF, 1), lambda i: (i, 0)),
            pl.BlockSpec((1, _BF), lambda i: (0, i)),
            pl.BlockSpec(memory_space=pltpu.VMEM),
        ],
        out_specs=(
            pl.BlockSpec((_BF, F), lambda i: (i, 0)),
            pl.BlockSpec((1, F), lambda i: (0, 0)),
        ),
        compiler_params=pltpu.CompilerParams(
            dimension_semantics=("arbitrary",),
            vmem_limit_bytes=60000 * 1024,
        ),
        name="lora_fold",
    )(kernel, lora_a, lora_b, scale.reshape(H, 1), ln_bias.reshape(1, H),
      bias.reshape(1, F))

    n_tiles = M // _BM
    x2 = x.reshape(M, H)

    out = pl.pallas_call(
        _main_body,
        out_shape=jax.ShapeDtypeStruct((M, F), jnp.float32),
        grid=(n_tiles + 1,),
        in_specs=[
            pl.BlockSpec((_BM, H), lambda i: (jnp.minimum(i, n_tiles - 1), 0)),
            pl.BlockSpec(memory_space=pltpu.VMEM),
            pl.BlockSpec(memory_space=pltpu.VMEM),
        ],
        out_specs=pl.BlockSpec((_BM, F),
                               lambda i: (jnp.maximum(i - 1, 0), 0)),
        scratch_shapes=[pltpu.VMEM((2, _BM, H), jnp.bfloat16)],
        compiler_params=pltpu.CompilerParams(
            dimension_semantics=("arbitrary",),
            vmem_limit_bytes=60000 * 1024,
        ),
        name="ln_dense",
    )(x2, w_folded, row_bias)
    return out.reshape(B, S, F)
